# baseline (speedup 1.0000x reference)
import functools

import jax
import jax.numpy as jnp
import numpy as np
from jax import lax
from jax.experimental import pallas as pl
from jax.experimental.pallas import tpu as pltpu
from jax.experimental.pallas import tpu_sc as plsc

N = 10000
E = 160000
IN_NEIGH = 32
IN_SELF = 32
OUT = 16
EDGE_DIM = 16
EDGE_HID = 64
BN_EPS = 1e-5

NW = 32
EPW = E // NW
CH = 125
JC = EPW // CH
CPB = 8
NBLK = JC // CPB
NPS = 624

def _gather_body(table_hbm, idx_hbm, out_hbm, idx_v, rows_v, sem):
    cid = lax.axis_index("c")
    sid = lax.axis_index("s")
    wid = sid * 2 + cid
    pltpu.sync_copy(idx_hbm.at[pl.ds(wid * JC, JC)], idx_v)

    def block(b, carry):
        copies = [
            pltpu.async_copy(table_hbm.at[idx_v.at[b * CPB + j]],
                             rows_v.at[pl.ds(j * CH, CH)], sem)
            for j in range(CPB)
        ]
        for c in copies:
            c.wait()
        pltpu.sync_copy(rows_v,
                        out_hbm.at[pl.ds(wid * EPW + b * CPB * CH, CPB * CH)])
        return carry

    lax.fori_loop(0, NBLK, block, 0)


@functools.lru_cache(maxsize=None)
def _sc_gather():
    return pl.kernel(
        _gather_body,
        out_type=jax.ShapeDtypeStruct((E, IN_NEIGH), jnp.float32),
        mesh=plsc.VectorSubcoreMesh(core_axis_name="c", subcore_axis_name="s"),
        scratch_types=[
            pltpu.VMEM((JC, CH), jnp.int32),
            pltpu.VMEM((CPB * CH, IN_NEIGH), jnp.float32),
            pltpu.SemaphoreType.DMA,
        ],
        compiler_params=pltpu.CompilerParams(use_tc_tiling_on_sc=False),
    )


def _scatter_body(msg_hbm, dst_hbm, out_hbm, acc_sh, idx_v, msg_v, zero_v):
    cid = lax.axis_index("c")
    sid = lax.axis_index("s")
    wid = sid * 2 + cid

    def zrow(i, carry):
        zero_v[i, :] = jnp.zeros((16,), jnp.float32)
        return carry

    lax.fori_loop(0, NPS + 16, zrow, 0)
    pltpu.sync_copy(zero_v.at[pl.ds(0, NPS)], acc_sh.at[pl.ds(sid * NPS, NPS)])

    @pl.when(sid == 15)
    def _zero_tail():
        pltpu.sync_copy(zero_v.at[pl.ds(0, N - 16 * NPS)],
                        acc_sh.at[pl.ds(16 * NPS, N - 16 * NPS)])

    pltpu.sync_copy(dst_hbm.at[pl.ds(wid * JC, JC)], idx_v)
    pltpu.sync_copy(msg_hbm.at[pl.ds(wid * EPW, EPW)], msg_v)
    plsc.subcore_barrier()

    def chunk(j, carry):
        pltpu.sync_copy(msg_v.at[pl.ds(j * CH, CH)], acc_sh.at[idx_v.at[j]],
                        add=True)
        return carry

    lax.fori_loop(0, JC, chunk, 0)
    plsc.subcore_barrier()
    pltpu.sync_copy(acc_sh.at[pl.ds(sid * NPS, NPS)],
                    out_hbm.at[cid].at[pl.ds(sid * NPS, NPS)])

    @pl.when(sid == 15)
    def _out_tail():
        pltpu.sync_copy(acc_sh.at[pl.ds(16 * NPS, N - 16 * NPS)],
                        out_hbm.at[cid].at[pl.ds(16 * NPS, N - 16 * NPS)])


@functools.lru_cache(maxsize=None)
def _sc_scatter():
    return pl.kernel(
        _scatter_body,
        out_type=jax.ShapeDtypeStruct((2, N, OUT), jnp.float32),
        mesh=plsc.VectorSubcoreMesh(core_axis_name="c", subcore_axis_name="s"),
        scratch_types=[
            pltpu.VMEM_SHARED((N, OUT), jnp.float32),
            pltpu.VMEM((JC, CH), jnp.int32),
            pltpu.VMEM((EPW, OUT), jnp.float32),
            pltpu.VMEM((NPS + 16, OUT), jnp.float32),
        ],
        compiler_params=pltpu.CompilerParams(use_tc_tiling_on_sc=False),
    )


BE = 3200


def _edge_body(efp_ref, h4_ref, w1j_ref, be1_ref, we2_ref, be2_ref,
               repq_ref, selp_ref, out_ref):
    efp = efp_ref[...]
    h4 = h4_ref[...]
    q8 = BE // 8
    acc = None
    for q in range(4):
        hexp_q = jnp.dot(h4, repq_ref[q], preferred_element_type=jnp.float32)
        for h in range(2):
            j = 2 * q + h
            hmid_j = jnp.dot(efp, w1j_ref[j],
                             preferred_element_type=jnp.float32) + be1_ref[...]
            hmid_j = jnp.maximum(hmid_j, 0.0)
            ew_j = jnp.dot(hmid_j, we2_ref[...],
                           preferred_element_type=jnp.float32) + be2_ref[...]
            prod_j = hexp_q[h * q8:(h + 1) * q8] * ew_j
            part = jnp.dot(prod_j, selp_ref[j],
                           preferred_element_type=jnp.float32)
            acc = part if acc is None else acc + part
    out_ref[...] = acc


def _tc_edge(efp, h_src4, W1j, be1, We2, be2, repq, selp):
    grid = (E // BE,)
    return pl.pallas_call(
        _edge_body,
        grid=grid,
        in_specs=[
            pl.BlockSpec((BE // 8, 128), lambda i: (i, 0)),
            pl.BlockSpec((BE // 4, 128), lambda i: (i, 0)),
            pl.BlockSpec((8, 128, EDGE_HID), lambda i: (0, 0, 0)),
            pl.BlockSpec((1, EDGE_HID), lambda i: (0, 0)),
            pl.BlockSpec((EDGE_HID, IN_NEIGH * OUT), lambda i: (0, 0)),
            pl.BlockSpec((1, IN_NEIGH * OUT), lambda i: (0, 0)),
            pl.BlockSpec((4, 128, IN_NEIGH * OUT), lambda i: (0, 0, 0)),
            pl.BlockSpec((8, IN_NEIGH * OUT, 128), lambda i: (0, 0, 0)),
        ],
        out_specs=pl.BlockSpec((BE // 8, 128), lambda i: (i, 0)),
        out_shape=jax.ShapeDtypeStruct((E // 8, 128), jnp.float32),
        compiler_params=pltpu.CompilerParams(
            dimension_semantics=("arbitrary",)),
    )(efp, h_src4, W1j, be1, We2, be2, repq, selp)


def _final_body(hs_ref, ws_ref, g_ref, b_ref, p_ref, out_ref):
    s = jnp.dot(hs_ref[...], ws_ref[...], preferred_element_type=jnp.float32)
    mean = jnp.mean(s, axis=0, keepdims=True)
    var = jnp.mean((s - mean) ** 2, axis=0, keepdims=True)
    sn = (s - mean) * lax.rsqrt(var + BN_EPS) * g_ref[...] + b_ref[...]
    t = jnp.tanh(sn)
    hn = p_ref[0] + p_ref[1]
    z = jnp.maximum(t + hn, 0.0)
    nrm = jnp.sqrt(jnp.sum(z * z, axis=1, keepdims=True))
    nrm = jnp.where(nrm == 0.0, 1.0, nrm)
    out_ref[...] = z / nrm


def _tc_final(h_self, W_self, bn_gamma, bn_beta, partial):
    return pl.pallas_call(
        _final_body,
        out_shape=jax.ShapeDtypeStruct((N, OUT), jnp.float32),
    )(h_self, W_self, bn_gamma.reshape(1, OUT), bn_beta.reshape(1, OUT),
      partial)


_REPQ = np.zeros((4, 128, IN_NEIGH * OUT), np.float32)
for _q in range(4):
    for _i in range(IN_NEIGH):
        _REPQ[_q, _q * IN_NEIGH + _i, _i * OUT:(_i + 1) * OUT] = 1.0
_SELP = np.zeros((8, IN_NEIGH * OUT, 128), np.float32)
for _j in range(8):
    for _c in range(IN_NEIGH * OUT):
        _SELP[_j, _c, _j * OUT + (_c % OUT)] = 1.0
_W1M = np.zeros((8, 128, EDGE_DIM), np.float32)
for _j in range(8):
    for _c in range(EDGE_DIM):
        _W1M[_j, _j * EDGE_DIM + _c, _c] = 1.0


def kernel(h_neigh, h_self, edge_features, edge_index, W_self, bn_gamma,
           bn_beta, We1, be1, We2, be2):
    nb = E // BE
    src2d = (edge_index[0].reshape(nb, BE // 8, 4, 2).transpose(0, 3, 1, 2)
             .reshape(E // CH, CH))
    dst2d = edge_index[1].reshape(E // CH, CH)
    h_src = _sc_gather()(h_neigh, src2d)
    W1j = jnp.asarray(_W1M) @ We1
    msg = _tc_edge(edge_features.reshape(E // 8, 128),
                   h_src.reshape(E // 4, 128),
                   W1j, be1.reshape(1, EDGE_HID),
                   We2, be2.reshape(1, IN_NEIGH * OUT),
                   jnp.asarray(_REPQ), jnp.asarray(_SELP))
    partial = _sc_scatter()(msg.reshape(E, OUT), dst2d)
    return _tc_final(h_self, W_self, bn_gamma, bn_beta, partial)


# speedup vs baseline: 1.7973x; 1.7973x over previous
import functools

import jax
import jax.numpy as jnp
import numpy as np
from jax import lax
from jax.experimental import pallas as pl
from jax.experimental.pallas import tpu as pltpu
from jax.experimental.pallas import tpu_sc as plsc

N = 10000
E = 160000
IN_NEIGH = 32
IN_SELF = 32
OUT = 16
EDGE_DIM = 16
EDGE_HID = 64
BN_EPS = 1e-5

NW = 32
EPW = E // NW
CH = 125
JC = EPW // CH
CPB = 8
NBLK = JC // CPB
NPS = 624

def _gather_body(table_hbm, idx_hbm, out_hbm, idx_v, rows_v, sem):
    cid = lax.axis_index("c")
    sid = lax.axis_index("s")
    wid = sid * 2 + cid
    pltpu.sync_copy(idx_hbm.at[pl.ds(wid * JC, JC)], idx_v)

    def block(b, carry):
        copies = [
            pltpu.async_copy(table_hbm.at[idx_v.at[b * CPB + j]],
                             rows_v.at[pl.ds(j * CH, CH)], sem)
            for j in range(CPB)
        ]
        for c in copies:
            c.wait()
        pltpu.sync_copy(rows_v,
                        out_hbm.at[pl.ds(wid * EPW + b * CPB * CH, CPB * CH)])
        return carry

    lax.fori_loop(0, NBLK, block, 0)


@functools.lru_cache(maxsize=None)
def _sc_gather():
    return pl.kernel(
        _gather_body,
        out_type=jax.ShapeDtypeStruct((E, IN_NEIGH), jnp.float32),
        mesh=plsc.VectorSubcoreMesh(core_axis_name="c", subcore_axis_name="s"),
        scratch_types=[
            pltpu.VMEM((JC, CH), jnp.int32),
            pltpu.VMEM((CPB * CH, IN_NEIGH), jnp.float32),
            pltpu.SemaphoreType.DMA,
        ],
        compiler_params=pltpu.CompilerParams(use_tc_tiling_on_sc=False),
    )


def _scatter_body(msg_hbm, dst_hbm, out_hbm, acc_sh, idx_v, msg_v, zero_v):
    cid = lax.axis_index("c")
    sid = lax.axis_index("s")
    wid = sid * 2 + cid

    def zrow(i, carry):
        zero_v[i, :] = jnp.zeros((16,), jnp.float32)
        return carry

    lax.fori_loop(0, NPS + 16, zrow, 0)
    pltpu.sync_copy(zero_v.at[pl.ds(0, NPS)], acc_sh.at[pl.ds(sid * NPS, NPS)])

    @pl.when(sid == 15)
    def _zero_tail():
        pltpu.sync_copy(zero_v.at[pl.ds(0, N - 16 * NPS)],
                        acc_sh.at[pl.ds(16 * NPS, N - 16 * NPS)])

    pltpu.sync_copy(dst_hbm.at[pl.ds(wid * JC, JC)], idx_v)
    pltpu.sync_copy(msg_hbm.at[pl.ds(wid * EPW, EPW)], msg_v)
    plsc.subcore_barrier()

    def chunk(j, carry):
        pltpu.sync_copy(msg_v.at[pl.ds(j * CH, CH)], acc_sh.at[idx_v.at[j]],
                        add=True)
        return carry

    lax.fori_loop(0, JC, chunk, 0)
    plsc.subcore_barrier()
    pltpu.sync_copy(acc_sh.at[pl.ds(sid * NPS, NPS)],
                    out_hbm.at[cid].at[pl.ds(sid * NPS, NPS)])

    @pl.when(sid == 15)
    def _out_tail():
        pltpu.sync_copy(acc_sh.at[pl.ds(16 * NPS, N - 16 * NPS)],
                        out_hbm.at[cid].at[pl.ds(16 * NPS, N - 16 * NPS)])


@functools.lru_cache(maxsize=None)
def _sc_scatter():
    return pl.kernel(
        _scatter_body,
        out_type=jax.ShapeDtypeStruct((2, N, OUT), jnp.float32),
        mesh=plsc.VectorSubcoreMesh(core_axis_name="c", subcore_axis_name="s"),
        scratch_types=[
            pltpu.VMEM_SHARED((N, OUT), jnp.float32),
            pltpu.VMEM((JC, CH), jnp.int32),
            pltpu.VMEM((EPW, OUT), jnp.float32),
            pltpu.VMEM((NPS + 16, OUT), jnp.float32),
        ],
        compiler_params=pltpu.CompilerParams(use_tc_tiling_on_sc=False),
    )


BE = 3200


def _edge_body(ef_ref, h4_ref, we1_ref, be1_ref, we2_ref, be2_ref,
               repq_ref, selp_ref, out_ref):
    hmid = jnp.dot(ef_ref[...], we1_ref[...],
                   preferred_element_type=jnp.float32) + be1_ref[...]
    hmid = jnp.maximum(hmid, 0.0)
    ew = jnp.dot(hmid, we2_ref[...],
                 preferred_element_type=jnp.float32) + be2_ref[...]
    h4 = h4_ref[...]
    q4 = BE // 4
    q8 = BE // 8
    acc = None
    for q in range(4):
        hexp_q = jnp.dot(h4, repq_ref[q], preferred_element_type=jnp.float32)
        prod_q = hexp_q * ew[q * q4:(q + 1) * q4]
        s_q = (prod_q[:, 0:128] + prod_q[:, 128:256]
               + prod_q[:, 256:384] + prod_q[:, 384:512])
        for h in range(2):
            j = 2 * q + h
            part = jnp.dot(s_q[h * q8:(h + 1) * q8], selp_ref[j],
                           preferred_element_type=jnp.float32)
            acc = part if acc is None else acc + part
    out_ref[...] = acc


def _tc_edge(ef, h_src4, We1, be1, We2, be2, repq, selp):
    grid = (E // BE,)
    return pl.pallas_call(
        _edge_body,
        grid=grid,
        in_specs=[
            pl.BlockSpec((BE, EDGE_DIM), lambda i: (i, 0)),
            pl.BlockSpec((BE // 4, 128), lambda i: (i, 0)),
            pl.BlockSpec((EDGE_DIM, EDGE_HID), lambda i: (0, 0)),
            pl.BlockSpec((1, EDGE_HID), lambda i: (0, 0)),
            pl.BlockSpec((EDGE_HID, IN_NEIGH * OUT), lambda i: (0, 0)),
            pl.BlockSpec((1, IN_NEIGH * OUT), lambda i: (0, 0)),
            pl.BlockSpec((4, 128, IN_NEIGH * OUT), lambda i: (0, 0, 0)),
            pl.BlockSpec((8, 128, 128), lambda i: (0, 0, 0)),
        ],
        out_specs=pl.BlockSpec((BE // 8, 128), lambda i: (i, 0)),
        out_shape=jax.ShapeDtypeStruct((E // 8, 128), jnp.float32),
        compiler_params=pltpu.CompilerParams(
            dimension_semantics=("arbitrary",)),
    )(ef, h_src4, We1, be1, We2, be2, repq, selp)


def _final_body(hs_ref, ws_ref, g_ref, b_ref, p_ref, out_ref):
    s = jnp.dot(hs_ref[...], ws_ref[...], preferred_element_type=jnp.float32)
    mean = jnp.mean(s, axis=0, keepdims=True)
    var = jnp.mean((s - mean) ** 2, axis=0, keepdims=True)
    sn = (s - mean) * lax.rsqrt(var + BN_EPS) * g_ref[...] + b_ref[...]
    t = jnp.tanh(sn)
    hn = p_ref[0] + p_ref[1]
    z = jnp.maximum(t + hn, 0.0)
    nrm = jnp.sqrt(jnp.sum(z * z, axis=1, keepdims=True))
    nrm = jnp.where(nrm == 0.0, 1.0, nrm)
    out_ref[...] = z / nrm


def _tc_final(h_self, W_self, bn_gamma, bn_beta, partial):
    return pl.pallas_call(
        _final_body,
        out_shape=jax.ShapeDtypeStruct((N, OUT), jnp.float32),
    )(h_self, W_self, bn_gamma.reshape(1, OUT), bn_beta.reshape(1, OUT),
      partial)


_REPQ = np.zeros((4, 128, IN_NEIGH * OUT), np.float32)
for _q in range(4):
    for _i in range(IN_NEIGH):
        _REPQ[_q, _q * IN_NEIGH + _i, _i * OUT:(_i + 1) * OUT] = 1.0
_SELP = np.zeros((8, 128, 128), np.float32)
for _j in range(8):
    for _c in range(128):
        _SELP[_j, _c, _j * OUT + (_c % OUT)] = 1.0


def kernel(h_neigh, h_self, edge_features, edge_index, W_self, bn_gamma,
           bn_beta, We1, be1, We2, be2):
    nb = E // BE
    src2d = (edge_index[0].reshape(nb, 4, BE // 4).transpose(0, 2, 1)
             .reshape(E // CH, CH))
    dst2d = (edge_index[1].reshape(nb, 8, BE // 8).transpose(0, 2, 1)
             .reshape(E // CH, CH))
    h_src = _sc_gather()(h_neigh, src2d)
    msg = _tc_edge(edge_features, h_src.reshape(E // 4, 128),
                   We1, be1.reshape(1, EDGE_HID),
                   We2, be2.reshape(1, IN_NEIGH * OUT),
                   jnp.asarray(_REPQ), jnp.asarray(_SELP))
    partial = _sc_scatter()(msg.reshape(E, OUT), dst2d)
    return _tc_final(h_self, W_self, bn_gamma, bn_beta, partial)


# speedup vs baseline: 2.0735x; 1.1536x over previous
import functools

import jax
import jax.numpy as jnp
import numpy as np
from jax import lax
from jax.experimental import pallas as pl
from jax.experimental.pallas import tpu as pltpu
from jax.experimental.pallas import tpu_sc as plsc

N = 10000
E = 160000
IN_NEIGH = 32
IN_SELF = 32
OUT = 16
EDGE_DIM = 16
EDGE_HID = 64
BN_EPS = 1e-5

NW = 32
EPW = E // NW
CH = 125
JC = EPW // CH
CPB = 8
NBLK = JC // CPB
NPS = 624

def _gather_body(table_hbm, idx_hbm, out_hbm, idx_v, rows_v, sem):
    cid = lax.axis_index("c")
    sid = lax.axis_index("s")
    wid = sid * 2 + cid
    pltpu.sync_copy(idx_hbm.at[pl.ds(wid * JC, JC)], idx_v)

    def block(b, carry):
        copies = [
            pltpu.async_copy(table_hbm.at[idx_v.at[b * CPB + j]],
                             rows_v.at[pl.ds(j * CH, CH)], sem)
            for j in range(CPB)
        ]
        for c in copies:
            c.wait()
        pltpu.sync_copy(rows_v,
                        out_hbm.at[pl.ds(wid * EPW + b * CPB * CH, CPB * CH)])
        return carry

    lax.fori_loop(0, NBLK, block, 0)


@functools.lru_cache(maxsize=None)
def _sc_gather():
    return pl.kernel(
        _gather_body,
        out_type=jax.ShapeDtypeStruct((E, IN_NEIGH), jnp.float32),
        mesh=plsc.VectorSubcoreMesh(core_axis_name="c", subcore_axis_name="s"),
        scratch_types=[
            pltpu.VMEM((JC, CH), jnp.int32),
            pltpu.VMEM((CPB * CH, IN_NEIGH), jnp.float32),
            pltpu.SemaphoreType.DMA,
        ],
        compiler_params=pltpu.CompilerParams(use_tc_tiling_on_sc=False),
    )


def _scatter_body(msg_hbm, dst_hbm, out_hbm, acc_sh, idx_v, msg_v, zero_v):
    cid = lax.axis_index("c")
    sid = lax.axis_index("s")
    wid = sid * 2 + cid

    def zrow(i, carry):
        zero_v[i, :] = jnp.zeros((16,), jnp.float32)
        return carry

    lax.fori_loop(0, NPS + 16, zrow, 0)
    pltpu.sync_copy(zero_v.at[pl.ds(0, NPS)], acc_sh.at[pl.ds(sid * NPS, NPS)])

    @pl.when(sid == 15)
    def _zero_tail():
        pltpu.sync_copy(zero_v.at[pl.ds(0, N - 16 * NPS)],
                        acc_sh.at[pl.ds(16 * NPS, N - 16 * NPS)])

    pltpu.sync_copy(dst_hbm.at[pl.ds(wid * JC, JC)], idx_v)
    pltpu.sync_copy(msg_hbm.at[pl.ds(wid * EPW, EPW)], msg_v)
    plsc.subcore_barrier()

    def chunk(j, carry):
        pltpu.sync_copy(msg_v.at[pl.ds(j * CH, CH)], acc_sh.at[idx_v.at[j]],
                        add=True)
        return carry

    lax.fori_loop(0, JC, chunk, 0)
    plsc.subcore_barrier()
    pltpu.sync_copy(acc_sh.at[pl.ds(sid * NPS, NPS)],
                    out_hbm.at[cid].at[pl.ds(sid * NPS, NPS)])

    @pl.when(sid == 15)
    def _out_tail():
        pltpu.sync_copy(acc_sh.at[pl.ds(16 * NPS, N - 16 * NPS)],
                        out_hbm.at[cid].at[pl.ds(16 * NPS, N - 16 * NPS)])


@functools.lru_cache(maxsize=None)
def _sc_scatter():
    return pl.kernel(
        _scatter_body,
        out_type=jax.ShapeDtypeStruct((2, N, OUT), jnp.float32),
        mesh=plsc.VectorSubcoreMesh(core_axis_name="c", subcore_axis_name="s"),
        scratch_types=[
            pltpu.VMEM_SHARED((N, OUT), jnp.float32),
            pltpu.VMEM((JC, CH), jnp.int32),
            pltpu.VMEM((EPW, OUT), jnp.float32),
            pltpu.VMEM((NPS + 16, OUT), jnp.float32),
        ],
        compiler_params=pltpu.CompilerParams(use_tc_tiling_on_sc=False),
    )


BE = 3200


def _edge_body(ef_ref, h4_ref, we1_ref, be1_ref, we2_ref, be2_ref,
               repq_ref, selp_ref, out_ref):
    hmid = lax.dot_general(ef_ref[...], we1_ref[...],
                           (((0,), (0,)), ((), ())),
                           preferred_element_type=jnp.float32) + be1_ref[...]
    hmid = jnp.maximum(hmid, 0.0)
    ew = jnp.dot(hmid, we2_ref[...],
                 preferred_element_type=jnp.float32) + be2_ref[...]
    h4 = h4_ref[...]
    q4 = BE // 4
    q8 = BE // 8
    acc = None
    for q in range(4):
        hexp_q = jnp.dot(h4, repq_ref[q], preferred_element_type=jnp.float32)
        prod_q = hexp_q * ew[q * q4:(q + 1) * q4]
        s_q = (prod_q[:, 0:128] + prod_q[:, 128:256]
               + prod_q[:, 256:384] + prod_q[:, 384:512])
        for h in range(2):
            j = 2 * q + h
            part = jnp.dot(s_q[h * q8:(h + 1) * q8], selp_ref[j],
                           preferred_element_type=jnp.float32)
            acc = part if acc is None else acc + part
    out_ref[...] = acc


def _tc_edge(ef, h_src4, We1, be1, We2, be2, repq, selp):
    grid = (E // BE,)
    return pl.pallas_call(
        _edge_body,
        grid=grid,
        in_specs=[
            pl.BlockSpec((EDGE_DIM, BE), lambda i: (0, i)),
            pl.BlockSpec((BE // 4, 128), lambda i: (i, 0)),
            pl.BlockSpec((EDGE_DIM, EDGE_HID), lambda i: (0, 0)),
            pl.BlockSpec((1, EDGE_HID), lambda i: (0, 0)),
            pl.BlockSpec((EDGE_HID, IN_NEIGH * OUT), lambda i: (0, 0)),
            pl.BlockSpec((1, IN_NEIGH * OUT), lambda i: (0, 0)),
            pl.BlockSpec((4, 128, IN_NEIGH * OUT), lambda i: (0, 0, 0)),
            pl.BlockSpec((8, 128, 128), lambda i: (0, 0, 0)),
        ],
        out_specs=pl.BlockSpec((BE // 8, 128), lambda i: (i, 0)),
        out_shape=jax.ShapeDtypeStruct((E // 8, 128), jnp.float32),
        compiler_params=pltpu.CompilerParams(
            dimension_semantics=("arbitrary",)),
    )(ef, h_src4, We1, be1, We2, be2, repq, selp)


def _final_body(hs_ref, ws_ref, g_ref, b_ref, p_ref, out_ref):
    s = jnp.dot(hs_ref[...], ws_ref[...], preferred_element_type=jnp.float32)
    mean = jnp.mean(s, axis=0, keepdims=True)
    var = jnp.mean((s - mean) ** 2, axis=0, keepdims=True)
    sn = (s - mean) * lax.rsqrt(var + BN_EPS) * g_ref[...] + b_ref[...]
    t = jnp.tanh(sn)
    hn = p_ref[0] + p_ref[1]
    z = jnp.maximum(t + hn, 0.0)
    nrm = jnp.sqrt(jnp.sum(z * z, axis=1, keepdims=True))
    nrm = jnp.where(nrm == 0.0, 1.0, nrm)
    out_ref[...] = z / nrm


def _tc_final(h_self, W_self, bn_gamma, bn_beta, partial):
    return pl.pallas_call(
        _final_body,
        out_shape=jax.ShapeDtypeStruct((N, OUT), jnp.float32),
    )(h_self, W_self, bn_gamma.reshape(1, OUT), bn_beta.reshape(1, OUT),
      partial)


_REPQ = np.zeros((4, 128, IN_NEIGH * OUT), np.float32)
for _q in range(4):
    for _i in range(IN_NEIGH):
        _REPQ[_q, _q * IN_NEIGH + _i, _i * OUT:(_i + 1) * OUT] = 1.0
_SELP = np.zeros((8, 128, 128), np.float32)
for _j in range(8):
    for _c in range(128):
        _SELP[_j, _c, _j * OUT + (_c % OUT)] = 1.0


def kernel(h_neigh, h_self, edge_features, edge_index, W_self, bn_gamma,
           bn_beta, We1, be1, We2, be2):
    nb = E // BE
    src2d = (edge_index[0].reshape(nb, 4, BE // 4).transpose(0, 2, 1)
             .reshape(E // CH, CH))
    dst2d = (edge_index[1].reshape(nb, 8, BE // 8).transpose(0, 2, 1)
             .reshape(E // CH, CH))
    h_src = _sc_gather()(h_neigh, src2d)
    msg = _tc_edge(edge_features.T, h_src.reshape(E // 4, 128),
                   We1, be1.reshape(1, EDGE_HID),
                   We2, be2.reshape(1, IN_NEIGH * OUT),
                   jnp.asarray(_REPQ), jnp.asarray(_SELP))
    partial = _sc_scatter()(msg.reshape(E, OUT), dst2d)
    return _tc_final(h_self, W_self, bn_gamma, bn_beta, partial)


# speedup vs baseline: 2.1573x; 1.0404x over previous
import functools

import jax
import jax.numpy as jnp
import numpy as np
from jax import lax
from jax.experimental import pallas as pl
from jax.experimental.pallas import tpu as pltpu
from jax.experimental.pallas import tpu_sc as plsc

N = 10000
E = 160000
IN_NEIGH = 32
IN_SELF = 32
OUT = 16
EDGE_DIM = 16
EDGE_HID = 64
BN_EPS = 1e-5

NW = 32
EPW = E // NW
CH = 125
JC = EPW // CH
CPB = 8
NBLK = JC // CPB
NPS = 624

def _gather_body(table_hbm, idx_hbm, out_hbm, idx_v, rows_v, sem):
    cid = lax.axis_index("c")
    sid = lax.axis_index("s")
    wid = sid * 2 + cid
    pltpu.sync_copy(idx_hbm.at[pl.ds(wid * JC, JC)], idx_v)

    def block(b, carry):
        copies = [
            pltpu.async_copy(table_hbm.at[idx_v.at[b * CPB + j]],
                             rows_v.at[pl.ds(j * CH, CH)], sem)
            for j in range(CPB)
        ]
        for c in copies:
            c.wait()
        pltpu.sync_copy(rows_v,
                        out_hbm.at[pl.ds(wid * EPW + b * CPB * CH, CPB * CH)])
        return carry

    lax.fori_loop(0, NBLK, block, 0)


@functools.lru_cache(maxsize=None)
def _sc_gather():
    return pl.kernel(
        _gather_body,
        out_type=jax.ShapeDtypeStruct((E, IN_NEIGH), jnp.float32),
        mesh=plsc.VectorSubcoreMesh(core_axis_name="c", subcore_axis_name="s"),
        scratch_types=[
            pltpu.VMEM((JC, CH), jnp.int32),
            pltpu.VMEM((CPB * CH, IN_NEIGH), jnp.float32),
            pltpu.SemaphoreType.DMA,
        ],
        compiler_params=pltpu.CompilerParams(use_tc_tiling_on_sc=False),
    )


def _scatter_body(msg_hbm, dst_hbm, out_hbm, acc_sh, idx_v, msg_v, zero_v):
    cid = lax.axis_index("c")
    sid = lax.axis_index("s")
    wid = sid * 2 + cid

    def zrow(i, carry):
        zero_v[i, :] = jnp.zeros((16,), jnp.float32)
        return carry

    lax.fori_loop(0, NPS + 16, zrow, 0)
    pltpu.sync_copy(zero_v.at[pl.ds(0, NPS)], acc_sh.at[pl.ds(sid * NPS, NPS)])

    @pl.when(sid == 15)
    def _zero_tail():
        pltpu.sync_copy(zero_v.at[pl.ds(0, N - 16 * NPS)],
                        acc_sh.at[pl.ds(16 * NPS, N - 16 * NPS)])

    pltpu.sync_copy(dst_hbm.at[pl.ds(wid * JC, JC)], idx_v)
    pltpu.sync_copy(msg_hbm.at[pl.ds(wid * EPW, EPW)], msg_v)
    plsc.subcore_barrier()

    def chunk(j, carry):
        pltpu.sync_copy(msg_v.at[pl.ds(j * CH, CH)], acc_sh.at[idx_v.at[j]],
                        add=True)
        return carry

    lax.fori_loop(0, JC, chunk, 0)
    plsc.subcore_barrier()
    pltpu.sync_copy(acc_sh.at[pl.ds(sid * NPS, NPS)],
                    out_hbm.at[cid].at[pl.ds(sid * NPS, NPS)])

    @pl.when(sid == 15)
    def _out_tail():
        pltpu.sync_copy(acc_sh.at[pl.ds(16 * NPS, N - 16 * NPS)],
                        out_hbm.at[cid].at[pl.ds(16 * NPS, N - 16 * NPS)])


@functools.lru_cache(maxsize=None)
def _sc_scatter():
    return pl.kernel(
        _scatter_body,
        out_type=jax.ShapeDtypeStruct((2, N, OUT), jnp.float32),
        mesh=plsc.VectorSubcoreMesh(core_axis_name="c", subcore_axis_name="s"),
        scratch_types=[
            pltpu.VMEM_SHARED((N, OUT), jnp.float32),
            pltpu.VMEM((JC, CH), jnp.int32),
            pltpu.VMEM((EPW, OUT), jnp.float32),
            pltpu.VMEM((NPS + 16, OUT), jnp.float32),
        ],
        compiler_params=pltpu.CompilerParams(use_tc_tiling_on_sc=False),
    )


BE = 6400


def _edge_body(ef_ref, h4_ref, we1_ref, be1_ref, we2_ref, be2_ref,
               repq_ref, selp_ref, out_ref):
    hmid = lax.dot_general(ef_ref[...], we1_ref[...],
                           (((0,), (0,)), ((), ())),
                           preferred_element_type=jnp.float32) + be1_ref[...]
    hmid = jnp.maximum(hmid, 0.0)
    ew = jnp.dot(hmid, we2_ref[...],
                 preferred_element_type=jnp.float32) + be2_ref[...]
    h4 = h4_ref[...]
    q4 = BE // 4
    q8 = BE // 8
    acc = None
    for q in range(4):
        hexp_q = jnp.dot(h4, repq_ref[q], preferred_element_type=jnp.float32)
        prod_q = hexp_q * ew[q * q4:(q + 1) * q4]
        s_q = (prod_q[:, 0:128] + prod_q[:, 128:256]
               + prod_q[:, 256:384] + prod_q[:, 384:512])
        for h in range(2):
            j = 2 * q + h
            part = jnp.dot(s_q[h * q8:(h + 1) * q8], selp_ref[j],
                           preferred_element_type=jnp.float32)
            acc = part if acc is None else acc + part
    out_ref[...] = acc


def _tc_edge(ef, h_src4, We1, be1, We2, be2, repq, selp):
    grid = (E // BE,)
    return pl.pallas_call(
        _edge_body,
        grid=grid,
        in_specs=[
            pl.BlockSpec((EDGE_DIM, BE), lambda i: (0, i)),
            pl.BlockSpec((BE // 4, 128), lambda i: (i, 0)),
            pl.BlockSpec((EDGE_DIM, EDGE_HID), lambda i: (0, 0)),
            pl.BlockSpec((1, EDGE_HID), lambda i: (0, 0)),
            pl.BlockSpec((EDGE_HID, IN_NEIGH * OUT), lambda i: (0, 0)),
            pl.BlockSpec((1, IN_NEIGH * OUT), lambda i: (0, 0)),
            pl.BlockSpec((4, 128, IN_NEIGH * OUT), lambda i: (0, 0, 0)),
            pl.BlockSpec((8, 128, 128), lambda i: (0, 0, 0)),
        ],
        out_specs=pl.BlockSpec((BE // 8, 128), lambda i: (i, 0)),
        out_shape=jax.ShapeDtypeStruct((E // 8, 128), jnp.float32),
        compiler_params=pltpu.CompilerParams(
            dimension_semantics=("arbitrary",)),
    )(ef, h_src4, We1, be1, We2, be2, repq, selp)


def _final_body(hs_ref, ws_ref, g_ref, b_ref, p_ref, out_ref):
    s = jnp.dot(hs_ref[...], ws_ref[...], preferred_element_type=jnp.float32)
    mean = jnp.mean(s, axis=0, keepdims=True)
    var = jnp.mean((s - mean) ** 2, axis=0, keepdims=True)
    sn = (s - mean) * lax.rsqrt(var + BN_EPS) * g_ref[...] + b_ref[...]
    t = jnp.tanh(sn)
    hn = p_ref[0] + p_ref[1]
    z = jnp.maximum(t + hn, 0.0)
    nrm = jnp.sqrt(jnp.sum(z * z, axis=1, keepdims=True))
    nrm = jnp.where(nrm == 0.0, 1.0, nrm)
    out_ref[...] = z / nrm


def _tc_final(h_self, W_self, bn_gamma, bn_beta, partial):
    return pl.pallas_call(
        _final_body,
        out_shape=jax.ShapeDtypeStruct((N, OUT), jnp.float32),
    )(h_self, W_self, bn_gamma.reshape(1, OUT), bn_beta.reshape(1, OUT),
      partial)


_REPQ = np.zeros((4, 128, IN_NEIGH * OUT), np.float32)
for _q in range(4):
    for _i in range(IN_NEIGH):
        _REPQ[_q, _q * IN_NEIGH + _i, _i * OUT:(_i + 1) * OUT] = 1.0
_SELP = np.zeros((8, 128, 128), np.float32)
for _j in range(8):
    for _c in range(128):
        _SELP[_j, _c, _j * OUT + (_c % OUT)] = 1.0


def kernel(h_neigh, h_self, edge_features, edge_index, W_self, bn_gamma,
           bn_beta, We1, be1, We2, be2):
    nb = E // BE
    src2d = (edge_index[0].reshape(nb, 4, BE // 4).transpose(0, 2, 1)
             .reshape(E // CH, CH))
    dst2d = (edge_index[1].reshape(nb, 8, BE // 8).transpose(0, 2, 1)
             .reshape(E // CH, CH))
    h_src = _sc_gather()(h_neigh, src2d)
    msg = _tc_edge(edge_features.T, h_src.reshape(E // 4, 128),
                   We1, be1.reshape(1, EDGE_HID),
                   We2, be2.reshape(1, IN_NEIGH * OUT),
                   jnp.asarray(_REPQ), jnp.asarray(_SELP))
    partial = _sc_scatter()(msg.reshape(E, OUT), dst2d)
    return _tc_final(h_self, W_self, bn_gamma, bn_beta, partial)


# speedup vs baseline: 2.2128x; 1.0258x over previous
import functools

import jax
import jax.numpy as jnp
import numpy as np
from jax import lax
from jax.experimental import pallas as pl
from jax.experimental.pallas import tpu as pltpu
from jax.experimental.pallas import tpu_sc as plsc

N = 10000
E = 160000
IN_NEIGH = 32
IN_SELF = 32
OUT = 16
EDGE_DIM = 16
EDGE_HID = 64
BN_EPS = 1e-5

NW = 32
EPW = E // NW
CH = 125
JC = EPW // CH
CPB = 8
NBLK = JC // CPB
NPS = 624

def _gather_body(table_hbm, idx_hbm, out_hbm, idx_v, rows_v, sem):
    cid = lax.axis_index("c")
    sid = lax.axis_index("s")
    wid = sid * 2 + cid
    pltpu.sync_copy(idx_hbm.at[pl.ds(wid * JC, JC)], idx_v)

    def block(b, carry):
        copies = [
            pltpu.async_copy(table_hbm.at[idx_v.at[b * CPB + j]],
                             rows_v.at[pl.ds(j * CH, CH)], sem)
            for j in range(CPB)
        ]
        for c in copies:
            c.wait()
        pltpu.sync_copy(rows_v,
                        out_hbm.at[pl.ds(wid * EPW + b * CPB * CH, CPB * CH)])
        return carry

    lax.fori_loop(0, NBLK, block, 0)


@functools.lru_cache(maxsize=None)
def _sc_gather():
    return pl.kernel(
        _gather_body,
        out_type=jax.ShapeDtypeStruct((E, IN_NEIGH), jnp.float32),
        mesh=plsc.VectorSubcoreMesh(core_axis_name="c", subcore_axis_name="s"),
        scratch_types=[
            pltpu.VMEM((JC, CH), jnp.int32),
            pltpu.VMEM((CPB * CH, IN_NEIGH), jnp.float32),
            pltpu.SemaphoreType.DMA,
        ],
        compiler_params=pltpu.CompilerParams(use_tc_tiling_on_sc=False),
    )


def _scatter_body(msg_hbm, dst_hbm, out_hbm, acc_sh, idx_v, msg_v, zero_v):
    cid = lax.axis_index("c")
    sid = lax.axis_index("s")
    wid = sid * 2 + cid

    def zrow(i, carry):
        zero_v[i, :] = jnp.zeros((16,), jnp.float32)
        return carry

    lax.fori_loop(0, NPS + 16, zrow, 0)
    pltpu.sync_copy(zero_v.at[pl.ds(0, NPS)], acc_sh.at[pl.ds(sid * NPS, NPS)])

    @pl.when(sid == 15)
    def _zero_tail():
        pltpu.sync_copy(zero_v.at[pl.ds(0, N - 16 * NPS)],
                        acc_sh.at[pl.ds(16 * NPS, N - 16 * NPS)])

    pltpu.sync_copy(dst_hbm.at[pl.ds(wid * JC, JC)], idx_v)
    pltpu.sync_copy(msg_hbm.at[pl.ds(wid * EPW, EPW)], msg_v)
    plsc.subcore_barrier()

    def chunk(j, carry):
        pltpu.sync_copy(msg_v.at[pl.ds(j * CH, CH)], acc_sh.at[idx_v.at[j]],
                        add=True)
        return carry

    lax.fori_loop(0, JC, chunk, 0)
    plsc.subcore_barrier()
    pltpu.sync_copy(acc_sh.at[pl.ds(sid * NPS, NPS)],
                    out_hbm.at[cid].at[pl.ds(sid * NPS, NPS)])

    @pl.when(sid == 15)
    def _out_tail():
        pltpu.sync_copy(acc_sh.at[pl.ds(16 * NPS, N - 16 * NPS)],
                        out_hbm.at[cid].at[pl.ds(16 * NPS, N - 16 * NPS)])


@functools.lru_cache(maxsize=None)
def _sc_scatter():
    return pl.kernel(
        _scatter_body,
        out_type=jax.ShapeDtypeStruct((2, N, OUT), jnp.float32),
        mesh=plsc.VectorSubcoreMesh(core_axis_name="c", subcore_axis_name="s"),
        scratch_types=[
            pltpu.VMEM_SHARED((N, OUT), jnp.float32),
            pltpu.VMEM((JC, CH), jnp.int32),
            pltpu.VMEM((EPW, OUT), jnp.float32),
            pltpu.VMEM((NPS + 16, OUT), jnp.float32),
        ],
        compiler_params=pltpu.CompilerParams(use_tc_tiling_on_sc=False),
    )


BE = 6400


def _edge_body(ef_ref, h4_ref, we1_ref, be1_ref, we2_ref, be2_ref,
               repq_ref, selp_ref, out_ref):
    hmid = lax.dot_general(ef_ref[...], we1_ref[...],
                           (((0,), (0,)), ((), ())),
                           preferred_element_type=jnp.float32) + be1_ref[...]
    hmid = jnp.maximum(hmid, 0.0)
    ew = jnp.dot(hmid, we2_ref[...],
                 preferred_element_type=jnp.float32) + be2_ref[...]
    h4 = h4_ref[...]
    q4 = BE // 4
    q8 = BE // 8
    acc = None
    for q in range(4):
        hexp_q = jnp.dot(h4, repq_ref[q], preferred_element_type=jnp.float32)
        prod_q = hexp_q * ew[q * q4:(q + 1) * q4]
        s_q = (prod_q[:, 0:128] + prod_q[:, 128:256]
               + prod_q[:, 256:384] + prod_q[:, 384:512])
        for h in range(2):
            j = 2 * q + h
            part = jnp.dot(s_q[h * q8:(h + 1) * q8], selp_ref[j],
                           preferred_element_type=jnp.float32)
            acc = part if acc is None else acc + part
    out_ref[...] = acc


def _tc_edge(ef, h_src4, We1, be1, We2, be2, repq, selp):
    grid = (E // BE,)
    return pl.pallas_call(
        _edge_body,
        grid=grid,
        in_specs=[
            pl.BlockSpec((EDGE_DIM, BE), lambda i: (0, i)),
            pl.BlockSpec((BE // 4, 128), lambda i: (i, 0)),
            pl.BlockSpec((EDGE_DIM, EDGE_HID), lambda i: (0, 0)),
            pl.BlockSpec((1, EDGE_HID), lambda i: (0, 0)),
            pl.BlockSpec((EDGE_HID, IN_NEIGH * OUT), lambda i: (0, 0)),
            pl.BlockSpec((1, IN_NEIGH * OUT), lambda i: (0, 0)),
            pl.BlockSpec((4, 128, IN_NEIGH * OUT), lambda i: (0, 0, 0)),
            pl.BlockSpec((8, 128, 128), lambda i: (0, 0, 0)),
        ],
        out_specs=pl.BlockSpec((BE // 8, 128), lambda i: (i, 0)),
        out_shape=jax.ShapeDtypeStruct((E // 8, 128), jnp.float32),
        compiler_params=pltpu.CompilerParams(
            dimension_semantics=("arbitrary",)),
    )(ef, h_src4, We1, be1, We2, be2, repq, selp)


def _final_body(hs_ref, ws_ref, g_ref, b_ref, p_ref, out_ref):
    s = jnp.dot(hs_ref[...], ws_ref[...], preferred_element_type=jnp.float32)
    mean = jnp.mean(s, axis=0, keepdims=True)
    var = jnp.mean((s - mean) ** 2, axis=0, keepdims=True)
    sn = (s - mean) * lax.rsqrt(var + BN_EPS) * g_ref[...] + b_ref[...]
    t = jnp.tanh(sn)
    hn = p_ref[0] + p_ref[1]
    z = jnp.maximum(t + hn, 0.0)
    nrm = jnp.sqrt(jnp.sum(z * z, axis=1, keepdims=True))
    nrm = jnp.where(nrm == 0.0, 1.0, nrm)
    out_ref[...] = z / nrm


def _tc_final(h_self, W_self, bn_gamma, bn_beta, partial):
    return pl.pallas_call(
        _final_body,
        out_shape=jax.ShapeDtypeStruct((N, OUT), jnp.float32),
    )(h_self, W_self, bn_gamma.reshape(1, OUT), bn_beta.reshape(1, OUT),
      partial)


_REPQ = np.zeros((4, 128, IN_NEIGH * OUT), np.float32)
for _q in range(4):
    for _i in range(IN_NEIGH):
        _REPQ[_q, _q * IN_NEIGH + _i, _i * OUT:(_i + 1) * OUT] = 1.0
_SELP = np.zeros((8, 128, 128), np.float32)
for _j in range(8):
    for _c in range(128):
        _SELP[_j, _c, _j * OUT + (_c % OUT)] = 1.0


_SRCPERM = (np.arange(E, dtype=np.int32).reshape(E // BE, 4, BE // 4)
            .transpose(0, 2, 1).reshape(E // CH, CH))


def kernel(h_neigh, h_self, edge_features, edge_index, W_self, bn_gamma,
           bn_beta, We1, be1, We2, be2):
    nb = E // BE
    src2d = jnp.take(edge_index[0], jnp.asarray(_SRCPERM), axis=0)
    dst2d = (edge_index[1].reshape(nb, 8, BE // 8).transpose(0, 2, 1)
             .reshape(E // CH, CH))
    h_src = _sc_gather()(h_neigh, src2d)
    msg = _tc_edge(edge_features.T, h_src.reshape(E // 4, 128),
                   We1, be1.reshape(1, EDGE_HID),
                   We2, be2.reshape(1, IN_NEIGH * OUT),
                   jnp.asarray(_REPQ), jnp.asarray(_SELP))
    partial = _sc_scatter()(msg.reshape(E, OUT), dst2d)
    return _tc_final(h_self, W_self, bn_gamma, bn_beta, partial)
